# Initial kernel scaffold; baseline (speedup 1.0000x reference)
import jax
import jax.numpy as jnp
from jax.experimental import pallas as pl


def kernel(nodes_feat, edge_index, edges_feat, nodes_num_norm_sqrt, edges_num_norm_sqrt, graph_ids, W_embed, b_embed, W0, b0, g0, be0, W1, b1, g1, be1, W2, b2, g2, be2, W_readout):
    raise NotImplementedError("write your pallas kernel here")


# speedup vs baseline: 6.7237x; 6.7237x over previous
import functools

import jax
import jax.numpy as jnp
from jax import lax
from jax.experimental import pallas as pl
from jax.experimental.pallas import tpu as pltpu
from jax.experimental.pallas import tpu_sc as plsc

N = 10000
E = 320000
D = 128
G = 32
EPS = 1e-5

NC = 2
NS = 16
NW = NC * NS
N2 = 10240
RPT = N2 // NS
C = 80
E_TILE = E // NW
NCHUNK = E_TILE // C
ZR = 32

def _sc_degrees_body(src_hbm, dst_hbm, dego_hbm, degi_hbm,
                     sidx, didx, ones_v, zbuf, acc_o, acc_i):
    c = lax.axis_index("c")
    s = lax.axis_index("s")
    wid = c * NS + s
    for i in range(C // 16):
        ones_v[pl.ds(i * 16, 16)] = jnp.ones((16,), jnp.float32)
    for i in range(RPT // 16):
        zbuf[pl.ds(i * 16, 16)] = jnp.zeros((16,), jnp.float32)
    row0 = pl.multiple_of(s * RPT, 8)
    pltpu.sync_copy(zbuf, acc_o.at[pl.ds(row0, RPT)])
    pltpu.sync_copy(zbuf, acc_i.at[pl.ds(row0, RPT)])
    plsc.subcore_barrier()
    base = wid * E_TILE

    def body(k, carry):
        off = pl.multiple_of(base + k * C, 8)
        pltpu.sync_copy(src_hbm.at[pl.ds(off, C)], sidx)
        pltpu.sync_copy(dst_hbm.at[pl.ds(off, C)], didx)
        pltpu.sync_copy(ones_v, acc_o.at[sidx], add=True)
        pltpu.sync_copy(ones_v, acc_i.at[didx], add=True)
        return carry

    lax.fori_loop(0, NCHUNK, body, 0)
    plsc.subcore_barrier()
    pltpu.sync_copy(acc_o.at[pl.ds(row0, RPT)], dego_hbm.at[c, pl.ds(row0, RPT)])
    pltpu.sync_copy(acc_i.at[pl.ds(row0, RPT)], degi_hbm.at[c, pl.ds(row0, RPT)])


def _sc_scatter_body(x_hbm, src_hbm, dst_hbm, out_hbm,
                     s0, d0, r0, s1, d1, r1, zbuf, acc, sem0, sem1):
    c = lax.axis_index("c")
    s = lax.axis_index("s")
    wid = c * NS + s
    for i in range(ZR):
        for j in range(D // 16):
            zbuf[i, pl.ds(j * 16, 16)] = jnp.zeros((16,), jnp.float32)
    row0 = pl.multiple_of(s * RPT, 8)

    def zbody(k, carry):
        pltpu.sync_copy(zbuf, acc.at[pl.ds(row0 + k * ZR, ZR)])
        return carry

    lax.fori_loop(0, RPT // ZR, zbody, 0)
    plsc.subcore_barrier()
    base = wid * E_TILE

    def fire(k, sv, dv, rv, sem):
        off = pl.multiple_of(base + k * C, 8)
        pltpu.sync_copy(src_hbm.at[pl.ds(off, C)], sv)
        pltpu.sync_copy(dst_hbm.at[pl.ds(off, C)], dv)
        pltpu.async_copy(x_hbm.at[sv], rv, sem)

    def drain(sv, dv, rv, sem):
        pltpu.make_async_copy(x_hbm.at[sv], rv, sem).wait()
        pltpu.sync_copy(rv, acc.at[dv], add=True)

    fire(0, s0, d0, r0, sem0)

    def body(i, carry):
        kk = i * 2
        fire(kk + 1, s1, d1, r1, sem1)
        drain(s0, d0, r0, sem0)
        fire(kk + 2, s0, d0, r0, sem0)
        drain(s1, d1, r1, sem1)
        return carry

    lax.fori_loop(0, (NCHUNK - 1) // 2, body, 0)
    drain(s0, d0, r0, sem0)
    plsc.subcore_barrier()
    pltpu.sync_copy(acc.at[pl.ds(row0, RPT)], out_hbm.at[c, pl.ds(row0, RPT)])


@functools.lru_cache(maxsize=None)
def _build_sc_kernels():
    mesh = plsc.VectorSubcoreMesh(core_axis_name="c", subcore_axis_name="s",
                                  num_cores=NC, num_subcores=NS)
    sc_degrees = pl.kernel(
        _sc_degrees_body,
        out_type=(
            jax.ShapeDtypeStruct((NC, N2), jnp.float32),
            jax.ShapeDtypeStruct((NC, N2), jnp.float32),
        ),
        mesh=mesh,
        scratch_types=[
            pltpu.VMEM((C,), jnp.int32),
            pltpu.VMEM((C,), jnp.int32),
            pltpu.VMEM((C,), jnp.float32),
            pltpu.VMEM((RPT,), jnp.float32),
            pltpu.VMEM_SHARED((N2,), jnp.float32),
            pltpu.VMEM_SHARED((N2,), jnp.float32),
        ],
    )
    sc_scatter = pl.kernel(
        _sc_scatter_body,
        out_type=jax.ShapeDtypeStruct((NC, N2, D), jnp.float32),
        mesh=mesh,
        scratch_types=[
            pltpu.VMEM((C,), jnp.int32),
            pltpu.VMEM((C,), jnp.int32),
            pltpu.VMEM((C, D), jnp.float32),
            pltpu.VMEM((C,), jnp.int32),
            pltpu.VMEM((C,), jnp.int32),
            pltpu.VMEM((C, D), jnp.float32),
            pltpu.VMEM((ZR, D), jnp.float32),
            pltpu.VMEM_SHARED((N2, D), jnp.float32),
            pltpu.SemaphoreType.DMA,
            pltpu.SemaphoreType.DMA,
        ],
    )
    return sc_degrees, sc_scatter


def _tc_embed_body(nf_ref, W_ref, b_ref, ns_ref, h_ref, x_ref):
    h = jnp.dot(nf_ref[...], W_ref[...], preferred_element_type=jnp.float32)
    h = h + b_ref[...]
    h_ref[...] = h
    x_ref[...] = h * ns_ref[...]


_tc_embed = pl.pallas_call(
    _tc_embed_body,
    out_shape=(
        jax.ShapeDtypeStruct((N, D), jnp.float32),
        jax.ShapeDtypeStruct((N, D), jnp.float32),
    ),
)


def _layer_core(p_ref, h_ref, W_ref, b_ref, g_ref, be_ref, sn_ref, nd_ref):
    agg = (p_ref[0, :N, :] + p_ref[1, :N, :]) * nd_ref[...]
    hn = jnp.dot(agg, W_ref[...], preferred_element_type=jnp.float32)
    hn = (hn + b_ref[...]) * sn_ref[...]
    mean = jnp.mean(hn, axis=0, keepdims=True)
    var = jnp.mean((hn - mean) ** 2, axis=0, keepdims=True)
    hn = g_ref[...] * (hn - mean) / jnp.sqrt(var + EPS) + be_ref[...]
    hn = jnp.maximum(hn, 0.0)
    return hn + h_ref[...]


def _tc_layer_body(p_ref, h_ref, W_ref, b_ref, g_ref, be_ref, sn_ref, nd_ref,
                   ns_ref, ho_ref, xo_ref):
    hnew = _layer_core(p_ref, h_ref, W_ref, b_ref, g_ref, be_ref, sn_ref, nd_ref)
    ho_ref[...] = hnew
    xo_ref[...] = hnew * ns_ref[...]


_tc_layer = pl.pallas_call(
    _tc_layer_body,
    out_shape=(
        jax.ShapeDtypeStruct((N, D), jnp.float32),
        jax.ShapeDtypeStruct((N, D), jnp.float32),
    ),
)


def _tc_final_body(p_ref, h_ref, W_ref, b_ref, g_ref, be_ref, sn_ref, nd_ref,
                   gid_ref, Wr_ref, out_ref):
    hnew = _layer_core(p_ref, h_ref, W_ref, b_ref, g_ref, be_ref, sn_ref, nd_ref)
    onehot = (lax.broadcasted_iota(jnp.int32, (G, N), 0) == gid_ref[...]).astype(jnp.float32)
    counts = jnp.sum(onehot, axis=1, keepdims=True)
    hg = jnp.dot(onehot, hnew, preferred_element_type=jnp.float32)
    hg = hg / jnp.maximum(counts, 1.0)
    out_ref[...] = jnp.dot(hg, Wr_ref[...], preferred_element_type=jnp.float32)


_tc_final = pl.pallas_call(
    _tc_final_body,
    out_shape=jax.ShapeDtypeStruct((G, D), jnp.float32),
)


def kernel(nodes_feat, edge_index, edges_feat, nodes_num_norm_sqrt,
           edges_num_norm_sqrt, graph_ids,
           W_embed, b_embed, W0, b0, g0, be0, W1, b1, g1, be1,
           W2, b2, g2, be2, W_readout):
    src = edge_index[0]
    dst = edge_index[1]
    _sc_degrees, _sc_scatter = _build_sc_kernels()
    dego_p, degi_p = _sc_degrees(src, dst)
    deg_out = dego_p[0, :N] + dego_p[1, :N]
    deg_in = degi_p[0, :N] + degi_p[1, :N]
    nsrc = jnp.where(deg_out > 0, 1.0 / jnp.sqrt(jnp.maximum(deg_out, 1.0)), 0.0)
    ndst = jnp.where(deg_in > 0, 1.0 / jnp.sqrt(jnp.maximum(deg_in, 1.0)), 0.0)
    nsrc = nsrc.reshape(N, 1)
    ndst = ndst.reshape(N, 1)
    r2 = lambda v: v.reshape(1, D)

    h, x = _tc_embed(nodes_feat, W_embed, r2(b_embed), nsrc)
    for (W, b, g, be) in ((W0, b0, g0, be0), (W1, b1, g1, be1)):
        p = _sc_scatter(x, src, dst)
        h, x = _tc_layer(p, h, W, r2(b), r2(g), r2(be),
                         nodes_num_norm_sqrt, ndst, nsrc)
    p = _sc_scatter(x, src, dst)
    logits = _tc_final(p, h, W2, r2(b2), r2(g2), r2(be2),
                       nodes_num_norm_sqrt, ndst, graph_ids.reshape(1, N),
                       W_readout)
    return logits


# speedup vs baseline: 13.1509x; 1.9559x over previous
import functools

import jax
import jax.numpy as jnp
from jax import lax
from jax.experimental import pallas as pl
from jax.experimental.pallas import tpu as pltpu
from jax.experimental.pallas import tpu_sc as plsc

N = 10000
E = 320000
D = 128
G = 32
EPS = 1e-5

NC = 2
NS = 16
NW = NC * NS
N2 = 10240
RPT = N2 // NS
C = 80
E_TILE = E // NW
NCHUNK = E_TILE // C
ZR = 16

_DEG_LEAD = 8


def _sc_degrees_body(src_hbm, dst_hbm, dego_hbm, degi_hbm,
                     src_all, dst_all, ones_v, zbuf, acc_o, acc_i, sem, sem_i):
    c = lax.axis_index("c")
    s = lax.axis_index("s")
    wid = c * NS + s
    for i in range(C // 16):
        ones_v[pl.ds(i * 16, 16)] = jnp.ones((16,), jnp.float32)
    for i in range(RPT // 16):
        zbuf[pl.ds(i * 16, 16)] = jnp.zeros((16,), jnp.float32)
    row0 = pl.multiple_of(s * RPT, 8)
    pltpu.async_copy(src_hbm.at[pl.ds(wid * E_TILE, E_TILE)], src_all, sem_i)
    pltpu.async_copy(dst_hbm.at[pl.ds(wid * E_TILE, E_TILE)], dst_all, sem_i)
    pltpu.sync_copy(zbuf, acc_o.at[pl.ds(row0, RPT)])
    pltpu.sync_copy(zbuf, acc_i.at[pl.ds(row0, RPT)])
    pltpu.make_async_copy(src_hbm.at[pl.ds(0, E_TILE)], src_all, sem_i).wait()
    pltpu.make_async_copy(dst_hbm.at[pl.ds(0, E_TILE)], dst_all, sem_i).wait()
    plsc.subcore_barrier()

    def fire(k):
        off = pl.multiple_of(k * C, 8)
        pltpu.async_copy(ones_v, acc_o.at[src_all.at[pl.ds(off, C)]], sem,
                         add=True)
        pltpu.async_copy(ones_v, acc_i.at[dst_all.at[pl.ds(off, C)]], sem,
                         add=True)

    def drain_pair():
        pltpu.make_async_copy(ones_v, acc_o.at[pl.ds(0, C)], sem).wait()
        pltpu.make_async_copy(ones_v, acc_i.at[pl.ds(0, C)], sem).wait()

    for k in range(_DEG_LEAD):
        fire(k)

    def body(k, carry):
        fire(k)
        drain_pair()
        return carry

    lax.fori_loop(_DEG_LEAD, NCHUNK, body, 0)
    for _ in range(_DEG_LEAD):
        drain_pair()
    plsc.subcore_barrier()
    pltpu.sync_copy(acc_o.at[pl.ds(row0, RPT)], dego_hbm.at[c, pl.ds(row0, RPT)])
    pltpu.sync_copy(acc_i.at[pl.ds(row0, RPT)], degi_hbm.at[c, pl.ds(row0, RPT)])


NBUF = 3


def _sc_scatter_body(x_hbm, src_hbm, dst_hbm, out_hbm,
                     sidx, dst_all, rows_all, zbuf, acc, sem_i, sem_z,
                     sx, sg, ss):
    c = lax.axis_index("c")
    s = lax.axis_index("s")
    wid = c * NS + s
    for i in range(ZR):
        for j in range(D // 16):
            zbuf[i, pl.ds(j * 16, 16)] = jnp.zeros((16,), jnp.float32)
    row0 = pl.multiple_of(s * RPT, 8)
    base = wid * E_TILE
    pltpu.async_copy(dst_hbm.at[pl.ds(base, E_TILE)], dst_all, sem_i)
    for j in range(RPT // ZR):
        pltpu.async_copy(zbuf, acc.at[pl.ds(row0 + j * ZR, ZR)], sem_z)
    pltpu.make_async_copy(dst_hbm.at[pl.ds(0, E_TILE)], dst_all, sem_i).wait()
    for j in range(RPT // ZR):
        pltpu.make_async_copy(zbuf, acc.at[pl.ds(0, ZR)], sem_z).wait()
    plsc.subcore_barrier()

    def fire_idx(k, b):
        off = pl.multiple_of(base + k * C, 8)
        pltpu.async_copy(src_hbm.at[pl.ds(off, C)], sidx.at[b], sx.at[b])

    def wait_idx(b):
        pltpu.make_async_copy(src_hbm.at[pl.ds(0, C)], sidx.at[b],
                              sx.at[b]).wait()

    def fire_gather(k, b):
        pltpu.async_copy(x_hbm.at[sidx.at[b]], rows_all.at[b], sg.at[b])

    def wait_gather(b):
        pltpu.make_async_copy(x_hbm.at[pl.ds(0, C)], rows_all.at[b],
                              sg.at[b]).wait()

    def fire_scatter(k, b):
        off = pl.multiple_of(k * C, 8)
        pltpu.async_copy(rows_all.at[b], acc.at[dst_all.at[pl.ds(off, C)]],
                         ss.at[b], add=True)

    def wait_scatter(b):
        pltpu.make_async_copy(rows_all.at[b], acc.at[pl.ds(0, C)],
                              ss.at[b]).wait()

    def step(k, b, wait_s, do_g, do_i):
        bf = (b + NBUF - 1) % NBUF
        if wait_s:
            wait_scatter(bf)
        if do_g:
            wait_idx(bf)
            fire_gather(k + NBUF - 1, bf)
        wait_gather(b)
        if do_i:
            fire_idx(k + NBUF, b)
        fire_scatter(k, b)

    for b in range(NBUF):
        fire_idx(b, b)
    for b in range(NBUF - 1):
        wait_idx(b)
        fire_gather(b, b)
    for k in range(NBUF):
        step(k, k, k >= 1, True, True)

    def body(g, carry):
        k0 = g * NBUF
        for b in range(NBUF):
            step(k0 + b, b, True, True, True)
        return carry

    ntail = 5
    ngrp_main = (NCHUNK - ntail) // NBUF
    lax.fori_loop(1, ngrp_main, body, 0)
    for k in range(NCHUNK - ntail, NCHUNK):
        step(k, k % NBUF, True,
             k + NBUF - 1 <= NCHUNK - 1, k + NBUF <= NCHUNK - 1)
    wait_scatter((NCHUNK - 1) % NBUF)
    plsc.subcore_barrier()
    pltpu.sync_copy(acc.at[pl.ds(row0, RPT)], out_hbm.at[c, pl.ds(row0, RPT)])


@functools.lru_cache(maxsize=None)
def _build_sc_kernels():
    mesh = plsc.VectorSubcoreMesh(core_axis_name="c", subcore_axis_name="s",
                                  num_cores=NC, num_subcores=NS)
    sc_degrees = pl.kernel(
        _sc_degrees_body,
        out_type=(
            jax.ShapeDtypeStruct((NC, N2), jnp.float32),
            jax.ShapeDtypeStruct((NC, N2), jnp.float32),
        ),
        mesh=mesh,
        scratch_types=[
            pltpu.VMEM((E_TILE,), jnp.int32),
            pltpu.VMEM((E_TILE,), jnp.int32),
            pltpu.VMEM((C,), jnp.float32),
            pltpu.VMEM((RPT,), jnp.float32),
            pltpu.VMEM_SHARED((N2,), jnp.float32),
            pltpu.VMEM_SHARED((N2,), jnp.float32),
            pltpu.SemaphoreType.DMA,
            pltpu.SemaphoreType.DMA,
        ],
    )
    sc_scatter = pl.kernel(
        _sc_scatter_body,
        out_type=jax.ShapeDtypeStruct((NC, N2, D), jnp.float32),
        mesh=mesh,
        scratch_types=[
            pltpu.VMEM((NBUF, C), jnp.int32),
            pltpu.VMEM((E_TILE,), jnp.int32),
            pltpu.VMEM((NBUF, C, D), jnp.float32),
            pltpu.VMEM((ZR, D), jnp.float32),
            pltpu.VMEM_SHARED((N2, D), jnp.float32),
            pltpu.SemaphoreType.DMA,
            pltpu.SemaphoreType.DMA,
            pltpu.SemaphoreType.DMA((NBUF,)),
            pltpu.SemaphoreType.DMA((NBUF,)),
            pltpu.SemaphoreType.DMA((NBUF,)),
        ],
    )
    return sc_degrees, sc_scatter


def _tc_embed_body(nf_ref, W_ref, b_ref, ns_ref, h_ref, x_ref):
    h = jnp.dot(nf_ref[...], W_ref[...], preferred_element_type=jnp.float32)
    h = h + b_ref[...]
    h_ref[...] = h
    x_ref[...] = h * ns_ref[...]


_tc_embed = pl.pallas_call(
    _tc_embed_body,
    out_shape=(
        jax.ShapeDtypeStruct((N, D), jnp.float32),
        jax.ShapeDtypeStruct((N, D), jnp.float32),
    ),
)


def _layer_core(p_ref, h_ref, W_ref, b_ref, g_ref, be_ref, sn_ref, nd_ref):
    agg = (p_ref[0, :N, :] + p_ref[1, :N, :]) * nd_ref[...]
    hn = jnp.dot(agg, W_ref[...], preferred_element_type=jnp.float32)
    hn = (hn + b_ref[...]) * sn_ref[...]
    mean = jnp.mean(hn, axis=0, keepdims=True)
    var = jnp.mean((hn - mean) ** 2, axis=0, keepdims=True)
    hn = g_ref[...] * (hn - mean) / jnp.sqrt(var + EPS) + be_ref[...]
    hn = jnp.maximum(hn, 0.0)
    return hn + h_ref[...]


def _tc_layer_body(p_ref, h_ref, W_ref, b_ref, g_ref, be_ref, sn_ref, nd_ref,
                   ns_ref, ho_ref, xo_ref):
    hnew = _layer_core(p_ref, h_ref, W_ref, b_ref, g_ref, be_ref, sn_ref, nd_ref)
    ho_ref[...] = hnew
    xo_ref[...] = hnew * ns_ref[...]


_tc_layer = pl.pallas_call(
    _tc_layer_body,
    out_shape=(
        jax.ShapeDtypeStruct((N, D), jnp.float32),
        jax.ShapeDtypeStruct((N, D), jnp.float32),
    ),
)


def _tc_final_body(p_ref, h_ref, W_ref, b_ref, g_ref, be_ref, sn_ref, nd_ref,
                   gid_ref, Wr_ref, out_ref):
    hnew = _layer_core(p_ref, h_ref, W_ref, b_ref, g_ref, be_ref, sn_ref, nd_ref)
    onehot = (lax.broadcasted_iota(jnp.int32, (G, N), 0) == gid_ref[...]).astype(jnp.float32)
    counts = jnp.sum(onehot, axis=1, keepdims=True)
    hg = jnp.dot(onehot, hnew, preferred_element_type=jnp.float32)
    hg = hg / jnp.maximum(counts, 1.0)
    out_ref[...] = jnp.dot(hg, Wr_ref[...], preferred_element_type=jnp.float32)


_tc_final = pl.pallas_call(
    _tc_final_body,
    out_shape=jax.ShapeDtypeStruct((G, D), jnp.float32),
)


def kernel(nodes_feat, edge_index, edges_feat, nodes_num_norm_sqrt,
           edges_num_norm_sqrt, graph_ids,
           W_embed, b_embed, W0, b0, g0, be0, W1, b1, g1, be1,
           W2, b2, g2, be2, W_readout):
    _sc_degrees, _sc_scatter = _build_sc_kernels()
    dego_p, degi_p = _sc_degrees(edge_index[0], edge_index[1])
    deg_out = dego_p[0, :N] + dego_p[1, :N]
    deg_in = degi_p[0, :N] + degi_p[1, :N]
    nsrc = jnp.where(deg_out > 0, 1.0 / jnp.sqrt(jnp.maximum(deg_out, 1.0)), 0.0)
    ndst = jnp.where(deg_in > 0, 1.0 / jnp.sqrt(jnp.maximum(deg_in, 1.0)), 0.0)
    nsrc = nsrc.reshape(N, 1)
    ndst = ndst.reshape(N, 1)
    r2 = lambda v: v.reshape(1, D)

    h, x = _tc_embed(nodes_feat, W_embed, r2(b_embed), nsrc)
    for (W, b, g, be) in ((W0, b0, g0, be0), (W1, b1, g1, be1)):
        p = _sc_scatter(x, edge_index[0], edge_index[1])
        h, x = _tc_layer(p, h, W, r2(b), r2(g), r2(be),
                         nodes_num_norm_sqrt, ndst, nsrc)
    p = _sc_scatter(x, edge_index[0], edge_index[1])
    logits = _tc_final(p, h, W2, r2(b2), r2(g2), r2(be2),
                       nodes_num_norm_sqrt, ndst, graph_ids.reshape(1, N),
                       W_readout)
    return logits


# speedup vs baseline: 13.3153x; 1.0125x over previous
import functools

import jax
import jax.numpy as jnp
from jax import lax
from jax.experimental import pallas as pl
from jax.experimental.pallas import tpu as pltpu
from jax.experimental.pallas import tpu_sc as plsc

N = 10000
E = 320000
D = 128
G = 32
EPS = 1e-5

NC = 2
NS = 16
NW = NC * NS
N2 = 10240
RPT = N2 // NS
C = 80
E_TILE = E // NW
NCHUNK = E_TILE // C
ZR = 16

_DEG_LEAD = 8


def _sc_degrees_body(src_hbm, dst_hbm, dego_hbm, degi_hbm,
                     src_all, dst_all, ones_v, zbuf, acc_o, acc_i, sem, sem_i):
    c = lax.axis_index("c")
    s = lax.axis_index("s")
    wid = c * NS + s
    for i in range(C // 16):
        ones_v[pl.ds(i * 16, 16)] = jnp.ones((16,), jnp.float32)
    for i in range(RPT // 16):
        zbuf[pl.ds(i * 16, 16)] = jnp.zeros((16,), jnp.float32)
    row0 = pl.multiple_of(s * RPT, 8)
    pltpu.async_copy(src_hbm.at[pl.ds(wid * E_TILE, E_TILE)], src_all, sem_i)
    pltpu.async_copy(dst_hbm.at[pl.ds(wid * E_TILE, E_TILE)], dst_all, sem_i)
    pltpu.sync_copy(zbuf, acc_o.at[pl.ds(row0, RPT)])
    pltpu.sync_copy(zbuf, acc_i.at[pl.ds(row0, RPT)])
    pltpu.make_async_copy(src_hbm.at[pl.ds(0, E_TILE)], src_all, sem_i).wait()
    pltpu.make_async_copy(dst_hbm.at[pl.ds(0, E_TILE)], dst_all, sem_i).wait()
    plsc.subcore_barrier()

    def fire(k):
        off = pl.multiple_of(k * C, 8)
        pltpu.async_copy(ones_v, acc_o.at[src_all.at[pl.ds(off, C)]], sem,
                         add=True)
        pltpu.async_copy(ones_v, acc_i.at[dst_all.at[pl.ds(off, C)]], sem,
                         add=True)

    def drain_pair():
        pltpu.make_async_copy(ones_v, acc_o.at[pl.ds(0, C)], sem).wait()
        pltpu.make_async_copy(ones_v, acc_i.at[pl.ds(0, C)], sem).wait()

    for k in range(_DEG_LEAD):
        fire(k)

    def body(k, carry):
        fire(k)
        drain_pair()
        return carry

    lax.fori_loop(_DEG_LEAD, NCHUNK, body, 0)
    for _ in range(_DEG_LEAD):
        drain_pair()
    plsc.subcore_barrier()
    pltpu.sync_copy(acc_o.at[pl.ds(row0, RPT)], dego_hbm.at[c, pl.ds(row0, RPT)])
    pltpu.sync_copy(acc_i.at[pl.ds(row0, RPT)], degi_hbm.at[c, pl.ds(row0, RPT)])


NBUF = 3


def _sc_scatter_body(x_hbm, src_hbm, dst_hbm, out_hbm,
                     sidx, dst_all, rows_all, zbuf, acc, sem_i, sem_z,
                     sx, sg, ss):
    c = lax.axis_index("c")
    s = lax.axis_index("s")
    wid = c * NS + s
    for i in range(ZR):
        for j in range(D // 16):
            zbuf[i, pl.ds(j * 16, 16)] = jnp.zeros((16,), jnp.float32)
    row0 = pl.multiple_of(s * RPT, 8)
    base = wid * E_TILE
    pltpu.async_copy(dst_hbm.at[pl.ds(base, E_TILE)], dst_all, sem_i)
    for j in range(RPT // ZR):
        pltpu.async_copy(zbuf, acc.at[pl.ds(row0 + j * ZR, ZR)], sem_z)
    pltpu.make_async_copy(dst_hbm.at[pl.ds(0, E_TILE)], dst_all, sem_i).wait()
    for j in range(RPT // ZR):
        pltpu.make_async_copy(zbuf, acc.at[pl.ds(0, ZR)], sem_z).wait()
    plsc.subcore_barrier()

    def fire_idx(k, b):
        off = pl.multiple_of(base + k * C, 8)
        pltpu.async_copy(src_hbm.at[pl.ds(off, C)], sidx.at[b], sx.at[b])

    def wait_idx(b):
        pltpu.make_async_copy(src_hbm.at[pl.ds(0, C)], sidx.at[b],
                              sx.at[b]).wait()

    def fire_gather(k, b):
        pltpu.async_copy(x_hbm.at[sidx.at[b]], rows_all.at[b], sg.at[b])

    def wait_gather(b):
        pltpu.make_async_copy(x_hbm.at[pl.ds(0, C)], rows_all.at[b],
                              sg.at[b]).wait()

    def fire_scatter(k, b):
        off = pl.multiple_of(k * C, 8)
        pltpu.async_copy(rows_all.at[b], acc.at[dst_all.at[pl.ds(off, C)]],
                         ss.at[b], add=True)

    def wait_scatter(b):
        pltpu.make_async_copy(rows_all.at[b], acc.at[pl.ds(0, C)],
                              ss.at[b]).wait()

    def step(k, b, wait_s, do_g, do_i):
        bf = (b + NBUF - 1) % NBUF
        if wait_s:
            wait_scatter(bf)
        if do_g:
            wait_idx(bf)
            fire_gather(k + NBUF - 1, bf)
        wait_gather(b)
        if do_i:
            fire_idx(k + NBUF, b)
        fire_scatter(k, b)

    for b in range(NBUF):
        fire_idx(b, b)
    for b in range(NBUF - 1):
        wait_idx(b)
        fire_gather(b, b)
    for k in range(NBUF):
        step(k, k, k >= 1, True, True)

    def body(g, carry):
        k0 = g * NBUF
        for b in range(NBUF):
            step(k0 + b, b, True, True, True)
        return carry

    ntail = 5
    ngrp_main = (NCHUNK - ntail) // NBUF
    lax.fori_loop(1, ngrp_main, body, 0)
    for k in range(NCHUNK - ntail, NCHUNK):
        step(k, k % NBUF, True,
             k + NBUF - 1 <= NCHUNK - 1, k + NBUF <= NCHUNK - 1)
    wait_scatter((NCHUNK - 1) % NBUF)
    plsc.subcore_barrier()
    pltpu.sync_copy(acc.at[pl.ds(row0, RPT)], out_hbm.at[c, pl.ds(row0, RPT)])


@functools.lru_cache(maxsize=None)
def _build_sc_kernels():
    mesh = plsc.VectorSubcoreMesh(core_axis_name="c", subcore_axis_name="s",
                                  num_cores=NC, num_subcores=NS)
    sc_degrees = pl.kernel(
        _sc_degrees_body,
        out_type=(
            jax.ShapeDtypeStruct((NC, N2), jnp.float32),
            jax.ShapeDtypeStruct((NC, N2), jnp.float32),
        ),
        mesh=mesh,
        scratch_types=[
            pltpu.VMEM((E_TILE,), jnp.int32),
            pltpu.VMEM((E_TILE,), jnp.int32),
            pltpu.VMEM((C,), jnp.float32),
            pltpu.VMEM((RPT,), jnp.float32),
            pltpu.VMEM_SHARED((N2,), jnp.float32),
            pltpu.VMEM_SHARED((N2,), jnp.float32),
            pltpu.SemaphoreType.DMA,
            pltpu.SemaphoreType.DMA,
        ],
    )
    sc_scatter = pl.kernel(
        _sc_scatter_body,
        out_type=jax.ShapeDtypeStruct((NC, N2, D), jnp.float32),
        mesh=mesh,
        scratch_types=[
            pltpu.VMEM((NBUF, C), jnp.int32),
            pltpu.VMEM((E_TILE,), jnp.int32),
            pltpu.VMEM((NBUF, C, D), jnp.float32),
            pltpu.VMEM((ZR, D), jnp.float32),
            pltpu.VMEM_SHARED((N2, D), jnp.float32),
            pltpu.SemaphoreType.DMA,
            pltpu.SemaphoreType.DMA,
            pltpu.SemaphoreType.DMA((NBUF,)),
            pltpu.SemaphoreType.DMA((NBUF,)),
            pltpu.SemaphoreType.DMA((NBUF,)),
        ],
    )
    return sc_degrees, sc_scatter


def _norm_from_deg(dp_ref):
    dg = dp_ref[0, :N] + dp_ref[1, :N]
    ns = jnp.where(dg > 0, 1.0 / jnp.sqrt(jnp.maximum(dg, 1.0)), 0.0)
    return ns.reshape(N, 1)


def _tc_embed_body(nf_ref, W_ref, b_ref, dego_ref, degi_ref,
                   h_ref, x_ref, ns_ref, nd_ref):
    h = jnp.dot(nf_ref[...], W_ref[...], preferred_element_type=jnp.float32)
    h = h + b_ref[...]
    ns = _norm_from_deg(dego_ref)
    nd = _norm_from_deg(degi_ref)
    h_ref[...] = h
    x_ref[...] = h * ns
    ns_ref[...] = ns
    nd_ref[...] = nd


_tc_embed = pl.pallas_call(
    _tc_embed_body,
    out_shape=(
        jax.ShapeDtypeStruct((N, D), jnp.float32),
        jax.ShapeDtypeStruct((N, D), jnp.float32),
        jax.ShapeDtypeStruct((N, 1), jnp.float32),
        jax.ShapeDtypeStruct((N, 1), jnp.float32),
    ),
)


def _layer_core(p_ref, h_ref, W_ref, b_ref, g_ref, be_ref, sn_ref, nd_ref):
    agg = (p_ref[0, :N, :] + p_ref[1, :N, :]) * nd_ref[...]
    hn = jnp.dot(agg, W_ref[...], preferred_element_type=jnp.float32)
    hn = (hn + b_ref[...]) * sn_ref[...]
    mean = jnp.mean(hn, axis=0, keepdims=True)
    var = jnp.mean((hn - mean) ** 2, axis=0, keepdims=True)
    hn = g_ref[...] * (hn - mean) / jnp.sqrt(var + EPS) + be_ref[...]
    hn = jnp.maximum(hn, 0.0)
    return hn + h_ref[...]


def _tc_layer_body(p_ref, h_ref, W_ref, b_ref, g_ref, be_ref, sn_ref, nd_ref,
                   ns_ref, ho_ref, xo_ref):
    hnew = _layer_core(p_ref, h_ref, W_ref, b_ref, g_ref, be_ref, sn_ref, nd_ref)
    ho_ref[...] = hnew
    xo_ref[...] = hnew * ns_ref[...]


_tc_layer = pl.pallas_call(
    _tc_layer_body,
    out_shape=(
        jax.ShapeDtypeStruct((N, D), jnp.float32),
        jax.ShapeDtypeStruct((N, D), jnp.float32),
    ),
)


def _tc_final_body(p_ref, h_ref, W_ref, b_ref, g_ref, be_ref, sn_ref, nd_ref,
                   gid_ref, Wr_ref, out_ref):
    hnew = _layer_core(p_ref, h_ref, W_ref, b_ref, g_ref, be_ref, sn_ref, nd_ref)
    onehot = (lax.broadcasted_iota(jnp.int32, (G, N), 0) == gid_ref[...]).astype(jnp.float32)
    counts = jnp.sum(onehot, axis=1, keepdims=True)
    hg = jnp.dot(onehot, hnew, preferred_element_type=jnp.float32)
    hg = hg / jnp.maximum(counts, 1.0)
    out_ref[...] = jnp.dot(hg, Wr_ref[...], preferred_element_type=jnp.float32)


_tc_final = pl.pallas_call(
    _tc_final_body,
    out_shape=jax.ShapeDtypeStruct((G, D), jnp.float32),
)


def kernel(nodes_feat, edge_index, edges_feat, nodes_num_norm_sqrt,
           edges_num_norm_sqrt, graph_ids,
           W_embed, b_embed, W0, b0, g0, be0, W1, b1, g1, be1,
           W2, b2, g2, be2, W_readout):
    _sc_degrees, _sc_scatter = _build_sc_kernels()
    dego_p, degi_p = _sc_degrees(edge_index[0], edge_index[1])
    r2 = lambda v: v.reshape(1, D)

    h, x, nsrc, ndst = _tc_embed(nodes_feat, W_embed, r2(b_embed),
                                 dego_p, degi_p)
    for (W, b, g, be) in ((W0, b0, g0, be0), (W1, b1, g1, be1)):
        p = _sc_scatter(x, edge_index[0], edge_index[1])
        h, x = _tc_layer(p, h, W, r2(b), r2(g), r2(be),
                         nodes_num_norm_sqrt, ndst, nsrc)
    p = _sc_scatter(x, edge_index[0], edge_index[1])
    logits = _tc_final(p, h, W2, r2(b2), r2(g2), r2(be2),
                       nodes_num_norm_sqrt, ndst, graph_ids.reshape(1, N),
                       W_readout)
    return logits
